# baseline (speedup 1.0000x reference)
import functools

import jax
import jax.numpy as jnp
from jax import lax
from jax.experimental import pallas as pl
from jax.experimental.pallas import tpu as pltpu
from jax.experimental.pallas import tpu_sc as plsc

N = 10000
E = 320000
D = 128
H = 64
NC = 2
NS = 16
NW = NC * NS
K = 80
NCH = 130
EPAD = NW * NCH * K
NB = 5
NBUF = 2 * NB
NPAD = 10240
RPT = NPAD // NS
RZ = 128
DPAD = NPAD
DZT = DPAD // NS
BR = 1024


def _proj2_body(x_ref, wa_ref, wb_ref, a_ref, b_ref):
    x = x_ref[...]
    a_ref[...] = jnp.dot(x, wa_ref[...], preferred_element_type=jnp.float32)
    b_ref[...] = jnp.dot(x, wb_ref[...], preferred_element_type=jnp.float32)


def _proj2(x, wa, wb):
    d = x.shape[1]
    return pl.pallas_call(
        _proj2_body,
        grid=((N + BR - 1) // BR,),
        in_specs=[
            pl.BlockSpec((BR, d), lambda i: (i, 0)),
            pl.BlockSpec((d, H), lambda i: (0, 0)),
            pl.BlockSpec((d, H), lambda i: (0, 0)),
        ],
        out_specs=[pl.BlockSpec((BR, H), lambda i: (i, 0))] * 2,
        out_shape=[jax.ShapeDtypeStruct((N, H), jnp.float32)] * 2,
    )(x, wa, wb)


def _combine2_body(s_ref, sp_ref, degp_ref, b_ref, wa_ref, wb_ref,
                   a_ref, b2_ref):
    deg = degp_ref[0] + degp_ref[1]
    rinv = 1.0 / jnp.maximum(deg, 1.0)
    s = sp_ref[0] + sp_ref[1]
    h = jnp.maximum(s_ref[...] + s * rinv[:, None] + b_ref[...], 0.0)
    a_ref[...] = jnp.dot(h, wa_ref[...], preferred_element_type=jnp.float32)
    b2_ref[...] = jnp.dot(h, wb_ref[...], preferred_element_type=jnp.float32)


def _combine2(hself, spart, degpart, bias, wa, wb):
    return pl.pallas_call(
        _combine2_body,
        grid=((N + BR - 1) // BR,),
        in_specs=[
            pl.BlockSpec((BR, H), lambda i: (i, 0)),
            pl.BlockSpec((NC, BR, H), lambda i: (0, i, 0)),
            pl.BlockSpec((NC, BR), lambda i: (0, i)),
            pl.BlockSpec((1, H), lambda i: (0, 0)),
            pl.BlockSpec((H, H), lambda i: (0, 0)),
            pl.BlockSpec((H, H), lambda i: (0, 0)),
        ],
        out_specs=[pl.BlockSpec((BR, H), lambda i: (i, 0))] * 2,
        out_shape=[jax.ShapeDtypeStruct((N, H), jnp.float32)] * 2,
    )(hself, spart, degpart, bias, wa, wb)


def _final_body(s_ref, sp_ref, degp_ref, b_ref, o_ref):
    deg = degp_ref[0] + degp_ref[1]
    rinv = 1.0 / jnp.maximum(deg, 1.0)
    s = sp_ref[0] + sp_ref[1]
    o_ref[...] = jnp.maximum(s_ref[...] + s * rinv[:, None] + b_ref[...], 0.0)


def _final(hself, spart, degpart, bias):
    return pl.pallas_call(
        _final_body,
        grid=((N + BR - 1) // BR,),
        in_specs=[
            pl.BlockSpec((BR, H), lambda i: (i, 0)),
            pl.BlockSpec((NC, BR, H), lambda i: (0, i, 0)),
            pl.BlockSpec((NC, BR), lambda i: (0, i)),
            pl.BlockSpec((1, H), lambda i: (0, 0)),
        ],
        out_specs=pl.BlockSpec((BR, H), lambda i: (i, 0)),
        out_shape=jax.ShapeDtypeStruct((N, H), jnp.float32),
    )(hself, spart, degpart, bias)


def _seg_sum_body(with_deg, *refs):
    if with_deg:
        (p_hbm, src_hbm, dst_hbm, out_hbm, deg_hbm,
         src_v, dst_v, rows_v, zbuf, zdeg,
         acc_sh, deg_sh, *sems) = refs
    else:
        (p_hbm, src_hbm, dst_hbm, out_hbm,
         src_v, dst_v, rows_v, zbuf, zdeg,
         acc_sh, deg_sh, *sems) = refs
    gsem = sems

    cid = lax.axis_index("c")
    sid = lax.axis_index("s")
    wid = cid * NS + sid

    def _zrow(r, c):
        for j in range(H // 16):
            zbuf[r, pl.ds(j * 16, 16)] = jnp.zeros((16,), jnp.float32)
        return c
    lax.fori_loop(0, RZ, _zrow, 0)
    for kk in range(RPT // RZ):
        pltpu.sync_copy(zbuf, acc_sh.at[pl.ds(sid * RPT + kk * RZ, RZ)])

    if with_deg:
        def _zd(r, c):
            zdeg[pl.ds(r * 16, 16)] = jnp.zeros((16,), jnp.float32)
            return c
        lax.fori_loop(0, DZT // 16, _zd, 0)
        pltpu.sync_copy(zdeg, deg_sh.at[pl.ds(sid * DZT, DZT)])
        for j in range(K // 16):
            zdeg[pl.ds(j * 16, 16)] = jnp.ones((16,), jnp.float32)

    pltpu.sync_copy(src_hbm.at[wid], src_v)
    pltpu.sync_copy(dst_hbm.at[wid], dst_v)

    pltpu.async_copy(p_hbm.at[src_v.at[0]], rows_v.at[0], gsem[0])

    plsc.subcore_barrier()

    def _group(gi, c):
        for u in range(2):
            j = gi * 2 + u
            pltpu.make_async_copy(p_hbm.at[pl.ds(0, K)], rows_v.at[u],
                                  gsem[u]).wait()

            @pl.when(j + 1 < NCH)
            def _():
                pltpu.async_copy(p_hbm.at[src_v.at[j + 1]], rows_v.at[1 - u],
                                 gsem[1 - u])
            pltpu.sync_copy(rows_v.at[u], acc_sh.at[dst_v.at[j]], add=True)
            if with_deg:
                pltpu.sync_copy(zdeg.at[pl.ds(0, K)],
                                deg_sh.at[dst_v.at[j]], add=True)
        return c
    lax.fori_loop(0, NCH // 2, _group, 0)

    plsc.subcore_barrier()

    pltpu.sync_copy(acc_sh.at[pl.ds(sid * RPT, RPT)],
                    out_hbm.at[pl.ds(cid * NPAD + sid * RPT, RPT)])
    if with_deg:
        pltpu.sync_copy(deg_sh.at[pl.ds(sid * DZT, DZT)],
                        deg_hbm.at[pl.ds(cid * DPAD + sid * DZT, DZT)])


def _make_seg_sum(with_deg):
    out_type = [jax.ShapeDtypeStruct((NC * NPAD, H), jnp.float32)]
    if with_deg:
        out_type.append(jax.ShapeDtypeStruct((NC * DPAD,), jnp.float32))
    else:
        out_type = out_type[0]
    return pl.kernel(
        functools.partial(_seg_sum_body, with_deg),
        out_type=out_type,
        mesh=plsc.VectorSubcoreMesh(core_axis_name="c", subcore_axis_name="s"),
        compiler_params=pltpu.CompilerParams(use_tc_tiling_on_sc=False),
        scratch_types=[
            pltpu.VMEM((NCH, K), jnp.int32),
            pltpu.VMEM((NCH, K), jnp.int32),
            pltpu.VMEM((2, K, H), jnp.float32),
            pltpu.VMEM((RZ, H), jnp.float32),
            pltpu.VMEM((DZT,), jnp.float32),
            pltpu.VMEM_SHARED((NPAD, H), jnp.float32),
            pltpu.VMEM_SHARED((DPAD,), jnp.float32),
        ] + [pltpu.SemaphoreType.DMA] * 2,
    )


_seg_sum_deg = _make_seg_sum(True)
_seg_sum = _make_seg_sum(False)


def kernel(feats, edge_index, W_self1, W_neigh1, b1, W_self2, W_neigh2, b2):
    npad_e = EPAD - E
    pad_dst = (jnp.arange(npad_e, dtype=jnp.int32) % (NPAD - N)) + N
    src = jnp.concatenate(
        [edge_index[0], jnp.zeros((npad_e,), jnp.int32)]).reshape(NW, NCH, K)
    dst = jnp.concatenate([edge_index[1], pad_dst]).reshape(NW, NCH, K)
    b1r = b1.reshape(1, H)
    b2r = b2.reshape(1, H)

    p1, self1 = _proj2(feats, W_neigh1, W_self1)
    s1_flat, deg_flat = _seg_sum_deg(p1, src, dst)
    s1 = s1_flat.reshape(NC, NPAD, H)
    degp = deg_flat.reshape(NC, DPAD)

    p2, self2 = _combine2(self1, s1, degp, b1r, W_neigh2, W_self2)
    s2_flat = _seg_sum(p2, src, dst)
    s2 = s2_flat.reshape(NC, NPAD, H)

    return _final(self2, s2, degp, b2r)


# speedup vs baseline: 2.6602x; 2.6602x over previous
import functools

import jax
import jax.numpy as jnp
from jax import lax
from jax.experimental import pallas as pl
from jax.experimental.pallas import tpu as pltpu
from jax.experimental.pallas import tpu_sc as plsc

N = 10000
E = 320000
D = 128
H = 64
NC = 2
NS = 16
NW = NC * NS
K = 80
NCH = 125
NPAD = 10240
RPT = NPAD // NS
RZ = 128
DPAD = NPAD
DZT = DPAD // NS
BR = 1024


def _proj2_body(x_ref, wa_ref, wb_ref, a_ref, b_ref):
    x = x_ref[...]
    a_ref[...] = jnp.dot(x, wa_ref[...], preferred_element_type=jnp.float32)
    b_ref[...] = jnp.dot(x, wb_ref[...], preferred_element_type=jnp.float32)


def _proj2(x, wa, wb):
    d = x.shape[1]
    return pl.pallas_call(
        _proj2_body,
        grid=((N + BR - 1) // BR,),
        in_specs=[
            pl.BlockSpec((BR, d), lambda i: (i, 0)),
            pl.BlockSpec((d, H), lambda i: (0, 0)),
            pl.BlockSpec((d, H), lambda i: (0, 0)),
        ],
        out_specs=[pl.BlockSpec((BR, H), lambda i: (i, 0))] * 2,
        out_shape=[jax.ShapeDtypeStruct((N, H), jnp.float32)] * 2,
    )(x, wa, wb)


def _combine2_body(s_ref, sp_ref, degp_ref, b_ref, wa_ref, wb_ref,
                   a_ref, b2_ref):
    deg = degp_ref[0] + degp_ref[1]
    rinv = 1.0 / jnp.maximum(deg, 1.0)
    s = sp_ref[0] + sp_ref[1]
    h = jnp.maximum(s_ref[...] + s * rinv[:, None] + b_ref[...], 0.0)
    a_ref[...] = jnp.dot(h, wa_ref[...], preferred_element_type=jnp.float32)
    b2_ref[...] = jnp.dot(h, wb_ref[...], preferred_element_type=jnp.float32)


def _combine2(hself, spart, degpart, bias, wa, wb):
    return pl.pallas_call(
        _combine2_body,
        grid=((N + BR - 1) // BR,),
        in_specs=[
            pl.BlockSpec((BR, H), lambda i: (i, 0)),
            pl.BlockSpec((NC, BR, H), lambda i: (0, i, 0)),
            pl.BlockSpec((NC, BR), lambda i: (0, i)),
            pl.BlockSpec((1, H), lambda i: (0, 0)),
            pl.BlockSpec((H, H), lambda i: (0, 0)),
            pl.BlockSpec((H, H), lambda i: (0, 0)),
        ],
        out_specs=[pl.BlockSpec((BR, H), lambda i: (i, 0))] * 2,
        out_shape=[jax.ShapeDtypeStruct((N, H), jnp.float32)] * 2,
    )(hself, spart, degpart, bias, wa, wb)


def _final_body(s_ref, sp_ref, degp_ref, b_ref, o_ref):
    deg = degp_ref[0] + degp_ref[1]
    rinv = 1.0 / jnp.maximum(deg, 1.0)
    s = sp_ref[0] + sp_ref[1]
    o_ref[...] = jnp.maximum(s_ref[...] + s * rinv[:, None] + b_ref[...], 0.0)


def _final(hself, spart, degpart, bias):
    return pl.pallas_call(
        _final_body,
        grid=((N + BR - 1) // BR,),
        in_specs=[
            pl.BlockSpec((BR, H), lambda i: (i, 0)),
            pl.BlockSpec((NC, BR, H), lambda i: (0, i, 0)),
            pl.BlockSpec((NC, BR), lambda i: (0, i)),
            pl.BlockSpec((1, H), lambda i: (0, 0)),
        ],
        out_specs=pl.BlockSpec((BR, H), lambda i: (i, 0)),
        out_shape=jax.ShapeDtypeStruct((N, H), jnp.float32),
    )(hself, spart, degpart, bias)


def _seg_sum_body(with_deg, *refs):
    if with_deg:
        (p_hbm, src_hbm, dst_hbm, out_hbm, deg_hbm,
         src_v, dst_v, rows_v, zbuf, zdeg,
         acc_sh, deg_sh, *sems) = refs
    else:
        (p_hbm, src_hbm, dst_hbm, out_hbm,
         src_v, dst_v, rows_v, zbuf, zdeg,
         acc_sh, deg_sh, *sems) = refs
    gsem = sems

    cid = lax.axis_index("c")
    sid = lax.axis_index("s")
    wid = cid * NS + sid

    def _zrow(r, c):
        for j in range(H // 16):
            zbuf[r, pl.ds(j * 16, 16)] = jnp.zeros((16,), jnp.float32)
        return c
    lax.fori_loop(0, RZ, _zrow, 0)
    for kk in range(RPT // RZ):
        pltpu.sync_copy(zbuf, acc_sh.at[pl.ds(sid * RPT + kk * RZ, RZ)])

    if with_deg:
        def _zd(r, c):
            zdeg[pl.ds(r * 16, 16)] = jnp.zeros((16,), jnp.float32)
            return c
        lax.fori_loop(0, DZT // 16, _zd, 0)
        pltpu.sync_copy(zdeg, deg_sh.at[pl.ds(sid * DZT, DZT)])
        for j in range(K // 16):
            zdeg[pl.ds(j * 16, 16)] = jnp.ones((16,), jnp.float32)

    pltpu.sync_copy(src_hbm.at[wid], src_v)
    pltpu.sync_copy(dst_hbm.at[wid], dst_v)

    pltpu.async_copy(p_hbm.at[src_v.at[0]], rows_v.at[0], gsem[0])

    plsc.subcore_barrier()

    def _group(gi, c):
        for u in range(2):
            j = gi * 2 + u
            pltpu.make_async_copy(p_hbm.at[pl.ds(0, K)], rows_v.at[u],
                                  gsem[u]).wait()

            @pl.when(j + 1 < NCH)
            def _():
                pltpu.async_copy(p_hbm.at[src_v.at[j + 1]], rows_v.at[1 - u],
                                 gsem[1 - u])
            pltpu.sync_copy(rows_v.at[u], acc_sh.at[dst_v.at[j]], add=True)
            if with_deg:
                pltpu.sync_copy(zdeg.at[pl.ds(0, K)],
                                deg_sh.at[dst_v.at[j]], add=True)
        return c
    lax.fori_loop(0, NCH // 2, _group, 0)

    pltpu.make_async_copy(p_hbm.at[pl.ds(0, K)], rows_v.at[0],
                          gsem[0]).wait()
    pltpu.sync_copy(rows_v.at[0], acc_sh.at[dst_v.at[NCH - 1]], add=True)
    if with_deg:
        pltpu.sync_copy(zdeg.at[pl.ds(0, K)],
                        deg_sh.at[dst_v.at[NCH - 1]], add=True)

    plsc.subcore_barrier()

    pltpu.sync_copy(acc_sh.at[pl.ds(sid * RPT, RPT)],
                    out_hbm.at[pl.ds(cid * NPAD + sid * RPT, RPT)])
    if with_deg:
        pltpu.sync_copy(deg_sh.at[pl.ds(sid * DZT, DZT)],
                        deg_hbm.at[pl.ds(cid * DPAD + sid * DZT, DZT)])


def _make_seg_sum(with_deg):
    out_type = [jax.ShapeDtypeStruct((NC * NPAD, H), jnp.float32)]
    if with_deg:
        out_type.append(jax.ShapeDtypeStruct((NC * DPAD,), jnp.float32))
    else:
        out_type = out_type[0]
    return pl.kernel(
        functools.partial(_seg_sum_body, with_deg),
        out_type=out_type,
        mesh=plsc.VectorSubcoreMesh(core_axis_name="c", subcore_axis_name="s"),
        compiler_params=pltpu.CompilerParams(use_tc_tiling_on_sc=False),
        scratch_types=[
            pltpu.VMEM((NCH, K), jnp.int32),
            pltpu.VMEM((NCH, K), jnp.int32),
            pltpu.VMEM((2, K, H), jnp.float32),
            pltpu.VMEM((RZ, H), jnp.float32),
            pltpu.VMEM((DZT,), jnp.float32),
            pltpu.VMEM_SHARED((NPAD, H), jnp.float32),
            pltpu.VMEM_SHARED((DPAD,), jnp.float32),
        ] + [pltpu.SemaphoreType.DMA] * 2,
    )


_seg_sum_deg = _make_seg_sum(True)
_seg_sum = _make_seg_sum(False)


def kernel(feats, edge_index, W_self1, W_neigh1, b1, W_self2, W_neigh2, b2):
    src = edge_index[0].reshape(NW, NCH, K)
    dst = edge_index[1].reshape(NW, NCH, K)
    b1r = b1.reshape(1, H)
    b2r = b2.reshape(1, H)

    p1, self1 = _proj2(feats, W_neigh1, W_self1)
    s1_flat, deg_flat = _seg_sum_deg(p1, src, dst)
    s1 = s1_flat.reshape(NC, NPAD, H)
    degp = deg_flat.reshape(NC, DPAD)

    p2, self2 = _combine2(self1, s1, degp, b1r, W_neigh2, W_self2)
    s2_flat = _seg_sum(p2, src, dst)
    s2 = s2_flat.reshape(NC, NPAD, H)

    return _final(self2, s2, degp, b2r)


# speedup vs baseline: 4.2168x; 1.5852x over previous
import functools

import jax
import jax.numpy as jnp
from jax import lax
from jax.experimental import pallas as pl
from jax.experimental.pallas import tpu as pltpu
from jax.experimental.pallas import tpu_sc as plsc

N = 10000
E = 320000
D = 128
H = 64
NC = 2
NS = 16
NW = NC * NS
K = 80
NCH = 125
NBUF = 5
GAHEAD = 4
NPAD = 10240
RPT = NPAD // NS
RZ = 128
DPAD = NPAD
DZT = DPAD // NS
BR = 1024


def _proj2_body(x_ref, wa_ref, wb_ref, a_ref, b_ref):
    x = x_ref[...]
    a_ref[...] = jnp.dot(x, wa_ref[...], preferred_element_type=jnp.float32)
    b_ref[...] = jnp.dot(x, wb_ref[...], preferred_element_type=jnp.float32)


def _proj2(x, wa, wb):
    d = x.shape[1]
    return pl.pallas_call(
        _proj2_body,
        grid=((N + BR - 1) // BR,),
        in_specs=[
            pl.BlockSpec((BR, d), lambda i: (i, 0)),
            pl.BlockSpec((d, H), lambda i: (0, 0)),
            pl.BlockSpec((d, H), lambda i: (0, 0)),
        ],
        out_specs=[pl.BlockSpec((BR, H), lambda i: (i, 0))] * 2,
        out_shape=[jax.ShapeDtypeStruct((N, H), jnp.float32)] * 2,
    )(x, wa, wb)


def _combine2_body(s_ref, sp_ref, degp_ref, b_ref, wa_ref, wb_ref,
                   a_ref, b2_ref):
    deg = degp_ref[0] + degp_ref[1]
    rinv = 1.0 / jnp.maximum(deg, 1.0)
    s = sp_ref[0] + sp_ref[1]
    h = jnp.maximum(s_ref[...] + s * rinv[:, None] + b_ref[...], 0.0)
    a_ref[...] = jnp.dot(h, wa_ref[...], preferred_element_type=jnp.float32)
    b2_ref[...] = jnp.dot(h, wb_ref[...], preferred_element_type=jnp.float32)


def _combine2(hself, spart, degpart, bias, wa, wb):
    return pl.pallas_call(
        _combine2_body,
        grid=((N + BR - 1) // BR,),
        in_specs=[
            pl.BlockSpec((BR, H), lambda i: (i, 0)),
            pl.BlockSpec((NC, BR, H), lambda i: (0, i, 0)),
            pl.BlockSpec((NC, BR), lambda i: (0, i)),
            pl.BlockSpec((1, H), lambda i: (0, 0)),
            pl.BlockSpec((H, H), lambda i: (0, 0)),
            pl.BlockSpec((H, H), lambda i: (0, 0)),
        ],
        out_specs=[pl.BlockSpec((BR, H), lambda i: (i, 0))] * 2,
        out_shape=[jax.ShapeDtypeStruct((N, H), jnp.float32)] * 2,
    )(hself, spart, degpart, bias, wa, wb)


def _final_body(s_ref, sp_ref, degp_ref, b_ref, o_ref):
    deg = degp_ref[0] + degp_ref[1]
    rinv = 1.0 / jnp.maximum(deg, 1.0)
    s = sp_ref[0] + sp_ref[1]
    o_ref[...] = jnp.maximum(s_ref[...] + s * rinv[:, None] + b_ref[...], 0.0)


def _final(hself, spart, degpart, bias):
    return pl.pallas_call(
        _final_body,
        grid=((N + BR - 1) // BR,),
        in_specs=[
            pl.BlockSpec((BR, H), lambda i: (i, 0)),
            pl.BlockSpec((NC, BR, H), lambda i: (0, i, 0)),
            pl.BlockSpec((NC, BR), lambda i: (0, i)),
            pl.BlockSpec((1, H), lambda i: (0, 0)),
        ],
        out_specs=pl.BlockSpec((BR, H), lambda i: (i, 0)),
        out_shape=jax.ShapeDtypeStruct((N, H), jnp.float32),
    )(hself, spart, degpart, bias)


def _seg_sum_body(with_deg, *refs):
    if with_deg:
        (p_hbm, src_hbm, dst_hbm, out_hbm, deg_hbm,
         src_v, dst_v, rows_v, zbuf, zdeg,
         acc_sh, deg_sh, *sems) = refs
    else:
        (p_hbm, src_hbm, dst_hbm, out_hbm,
         src_v, dst_v, rows_v, zbuf, zdeg,
         acc_sh, deg_sh, *sems) = refs
    gsem = sems

    cid = lax.axis_index("c")
    sid = lax.axis_index("s")
    wid = cid * NS + sid

    def _zrow(r, c):
        for j in range(H // 16):
            zbuf[r, pl.ds(j * 16, 16)] = jnp.zeros((16,), jnp.float32)
        return c
    lax.fori_loop(0, RZ, _zrow, 0)
    for kk in range(RPT // RZ):
        pltpu.sync_copy(zbuf, acc_sh.at[pl.ds(sid * RPT + kk * RZ, RZ)])

    if with_deg:
        def _zd(r, c):
            zdeg[pl.ds(r * 16, 16)] = jnp.zeros((16,), jnp.float32)
            return c
        lax.fori_loop(0, DZT // 16, _zd, 0)
        pltpu.sync_copy(zdeg, deg_sh.at[pl.ds(sid * DZT, DZT)])
        for j in range(K // 16):
            zdeg[pl.ds(j * 16, 16)] = jnp.ones((16,), jnp.float32)

    pltpu.sync_copy(src_hbm.at[wid], src_v)
    pltpu.sync_copy(dst_hbm.at[wid], dst_v)

    for b in range(GAHEAD):
        pltpu.async_copy(p_hbm.at[src_v.at[b]], rows_v.at[b], gsem[b])

    plsc.subcore_barrier()

    def _group(gi, c):
        for u in range(NBUF):
            j = gi * NBUF + u
            pltpu.make_async_copy(p_hbm.at[pl.ds(0, K)], rows_v.at[u],
                                  gsem[u]).wait()
            u2 = (u + GAHEAD) % NBUF

            @pl.when(j + GAHEAD < NCH)
            def _():
                pltpu.async_copy(p_hbm.at[src_v.at[j + GAHEAD]],
                                 rows_v.at[u2], gsem[u2])
            pltpu.sync_copy(rows_v.at[u], acc_sh.at[dst_v.at[j]], add=True)
            if with_deg:
                pltpu.sync_copy(zdeg.at[pl.ds(0, K)],
                                deg_sh.at[dst_v.at[j]], add=True)
        return c
    lax.fori_loop(0, NCH // NBUF, _group, 0)

    plsc.subcore_barrier()

    pltpu.sync_copy(acc_sh.at[pl.ds(sid * RPT, RPT)],
                    out_hbm.at[pl.ds(cid * NPAD + sid * RPT, RPT)])
    if with_deg:
        pltpu.sync_copy(deg_sh.at[pl.ds(sid * DZT, DZT)],
                        deg_hbm.at[pl.ds(cid * DPAD + sid * DZT, DZT)])


def _make_seg_sum(with_deg):
    out_type = [jax.ShapeDtypeStruct((NC * NPAD, H), jnp.float32)]
    if with_deg:
        out_type.append(jax.ShapeDtypeStruct((NC * DPAD,), jnp.float32))
    else:
        out_type = out_type[0]
    return pl.kernel(
        functools.partial(_seg_sum_body, with_deg),
        out_type=out_type,
        mesh=plsc.VectorSubcoreMesh(core_axis_name="c", subcore_axis_name="s"),
        compiler_params=pltpu.CompilerParams(use_tc_tiling_on_sc=False),
        scratch_types=[
            pltpu.VMEM((NCH, K), jnp.int32),
            pltpu.VMEM((NCH, K), jnp.int32),
            pltpu.VMEM((NBUF, K, H), jnp.float32),
            pltpu.VMEM((RZ, H), jnp.float32),
            pltpu.VMEM((DZT,), jnp.float32),
            pltpu.VMEM_SHARED((NPAD, H), jnp.float32),
            pltpu.VMEM_SHARED((DPAD,), jnp.float32),
        ] + [pltpu.SemaphoreType.DMA] * NBUF,
    )


_seg_sum_deg = _make_seg_sum(True)
_seg_sum = _make_seg_sum(False)


def kernel(feats, edge_index, W_self1, W_neigh1, b1, W_self2, W_neigh2, b2):
    src = edge_index[0].reshape(NW, NCH, K)
    dst = edge_index[1].reshape(NW, NCH, K)
    b1r = b1.reshape(1, H)
    b2r = b2.reshape(1, H)

    p1, self1 = _proj2(feats, W_neigh1, W_self1)
    s1_flat, deg_flat = _seg_sum_deg(p1, src, dst)
    s1 = s1_flat.reshape(NC, NPAD, H)
    degp = deg_flat.reshape(NC, DPAD)

    p2, self2 = _combine2(self1, s1, degp, b1r, W_neigh2, W_self2)
    s2_flat = _seg_sum(p2, src, dst)
    s2 = s2_flat.reshape(NC, NPAD, H)

    return _final(self2, s2, degp, b2r)


# speedup vs baseline: 4.2731x; 1.0134x over previous
import functools

import jax
import jax.numpy as jnp
from jax import lax
from jax.experimental import pallas as pl
from jax.experimental.pallas import tpu as pltpu
from jax.experimental.pallas import tpu_sc as plsc

N = 10000
E = 320000
D = 128
H = 64
NC = 2
NS = 16
NW = NC * NS
K = 80
NCH = 125
NBUF = 5
GAHEAD = 4
NPAD = 10240
RPT = NPAD // NS
RZ = 128
DPAD = NPAD
DZT = DPAD // NS
BR = 1024


def _proj2_body(x_ref, wa_ref, wb_ref, a_ref, b_ref):
    x = x_ref[...]
    a_ref[...] = jnp.dot(x, wa_ref[...], preferred_element_type=jnp.float32)
    b_ref[...] = jnp.dot(x, wb_ref[...], preferred_element_type=jnp.float32)


def _proj2(x, wa, wb):
    d = x.shape[1]
    return pl.pallas_call(
        _proj2_body,
        grid=((N + BR - 1) // BR,),
        in_specs=[
            pl.BlockSpec((BR, d), lambda i: (i, 0)),
            pl.BlockSpec((d, H), lambda i: (0, 0)),
            pl.BlockSpec((d, H), lambda i: (0, 0)),
        ],
        out_specs=[pl.BlockSpec((BR, H), lambda i: (i, 0))] * 2,
        out_shape=[jax.ShapeDtypeStruct((N, H), jnp.float32)] * 2,
    )(x, wa, wb)


def _combine2_body(s_ref, sp_ref, degp_ref, b_ref, wa_ref, wb_ref,
                   a_ref, b2_ref):
    deg = degp_ref[0] + degp_ref[1]
    rinv = 1.0 / jnp.maximum(deg, 1.0)
    s = sp_ref[0] + sp_ref[1]
    h = jnp.maximum(s_ref[...] + s * rinv[:, None] + b_ref[...], 0.0)
    a_ref[...] = jnp.dot(h, wa_ref[...], preferred_element_type=jnp.float32)
    b2_ref[...] = jnp.dot(h, wb_ref[...], preferred_element_type=jnp.float32)


def _combine2(hself, spart, degpart, bias, wa, wb):
    return pl.pallas_call(
        _combine2_body,
        grid=((N + BR - 1) // BR,),
        in_specs=[
            pl.BlockSpec((BR, H), lambda i: (i, 0)),
            pl.BlockSpec((NC, BR, H), lambda i: (0, i, 0)),
            pl.BlockSpec((NC, BR), lambda i: (0, i)),
            pl.BlockSpec((1, H), lambda i: (0, 0)),
            pl.BlockSpec((H, H), lambda i: (0, 0)),
            pl.BlockSpec((H, H), lambda i: (0, 0)),
        ],
        out_specs=[pl.BlockSpec((BR, H), lambda i: (i, 0))] * 2,
        out_shape=[jax.ShapeDtypeStruct((N, H), jnp.float32)] * 2,
    )(hself, spart, degpart, bias, wa, wb)


def _final_body(s_ref, sp_ref, degp_ref, b_ref, o_ref):
    deg = degp_ref[0] + degp_ref[1]
    rinv = 1.0 / jnp.maximum(deg, 1.0)
    s = sp_ref[0] + sp_ref[1]
    o_ref[...] = jnp.maximum(s_ref[...] + s * rinv[:, None] + b_ref[...], 0.0)


def _final(hself, spart, degpart, bias):
    return pl.pallas_call(
        _final_body,
        grid=((N + BR - 1) // BR,),
        in_specs=[
            pl.BlockSpec((BR, H), lambda i: (i, 0)),
            pl.BlockSpec((NC, BR, H), lambda i: (0, i, 0)),
            pl.BlockSpec((NC, BR), lambda i: (0, i)),
            pl.BlockSpec((1, H), lambda i: (0, 0)),
        ],
        out_specs=pl.BlockSpec((BR, H), lambda i: (i, 0)),
        out_shape=jax.ShapeDtypeStruct((N, H), jnp.float32),
    )(hself, spart, degpart, bias)


def _seg_sum_body(with_deg, *refs):
    if with_deg:
        (p_hbm, src_hbm, dst_hbm, out_hbm, deg_hbm,
         src_v, dst_v, rows_v, zbuf, zdeg,
         acc_sh, deg_sh, *sems) = refs
    else:
        (p_hbm, src_hbm, dst_hbm, out_hbm,
         src_v, dst_v, rows_v, zbuf, zdeg,
         acc_sh, deg_sh, *sems) = refs
    gsem = sems[:NBUF]
    ssem = sems[NBUF:2 * NBUF]
    dsem = sems[2 * NBUF]

    cid = lax.axis_index("c")
    sid = lax.axis_index("s")
    wid = cid * NS + sid

    def _zrow(r, c):
        for j in range(H // 16):
            zbuf[r, pl.ds(j * 16, 16)] = jnp.zeros((16,), jnp.float32)
        return c
    lax.fori_loop(0, RZ, _zrow, 0)
    for kk in range(RPT // RZ):
        pltpu.sync_copy(zbuf, acc_sh.at[pl.ds(sid * RPT + kk * RZ, RZ)])

    if with_deg:
        def _zd(r, c):
            zdeg[pl.ds(r * 16, 16)] = jnp.zeros((16,), jnp.float32)
            return c
        lax.fori_loop(0, DZT // 16, _zd, 0)
        pltpu.sync_copy(zdeg, deg_sh.at[pl.ds(sid * DZT, DZT)])
        for j in range(K // 16):
            zdeg[pl.ds(j * 16, 16)] = jnp.ones((16,), jnp.float32)

    pltpu.sync_copy(src_hbm.at[wid], src_v)
    pltpu.sync_copy(dst_hbm.at[wid], dst_v)

    for b in range(GAHEAD):
        pltpu.async_copy(p_hbm.at[src_v.at[b]], rows_v.at[b], gsem[b])

    plsc.subcore_barrier()

    def _group(gi, c):
        for u in range(NBUF):
            j = gi * NBUF + u
            pltpu.make_async_copy(p_hbm.at[pl.ds(0, K)], rows_v.at[u],
                                  gsem[u]).wait()
            u2 = (u + GAHEAD) % NBUF

            @pl.when(j >= 1)
            def _():
                pltpu.make_async_copy(rows_v.at[u2], acc_sh.at[pl.ds(0, K)],
                                      ssem[u2]).wait()
                if with_deg:
                    pltpu.make_async_copy(zdeg.at[pl.ds(0, K)],
                                          deg_sh.at[pl.ds(0, K)],
                                          dsem).wait()

            @pl.when(j + GAHEAD < NCH)
            def _():
                pltpu.async_copy(p_hbm.at[src_v.at[j + GAHEAD]],
                                 rows_v.at[u2], gsem[u2])
            pltpu.async_copy(rows_v.at[u], acc_sh.at[dst_v.at[j]],
                             ssem[u], add=True)
            if with_deg:
                pltpu.async_copy(zdeg.at[pl.ds(0, K)],
                                 deg_sh.at[dst_v.at[j]], dsem, add=True)
        return c
    lax.fori_loop(0, NCH // NBUF, _group, 0)

    pltpu.make_async_copy(rows_v.at[(NCH - 1) % NBUF],
                          acc_sh.at[pl.ds(0, K)],
                          ssem[(NCH - 1) % NBUF]).wait()
    if with_deg:
        pltpu.make_async_copy(zdeg.at[pl.ds(0, K)],
                              deg_sh.at[pl.ds(0, K)], dsem).wait()

    plsc.subcore_barrier()

    pltpu.sync_copy(acc_sh.at[pl.ds(sid * RPT, RPT)],
                    out_hbm.at[pl.ds(cid * NPAD + sid * RPT, RPT)])
    if with_deg:
        pltpu.sync_copy(deg_sh.at[pl.ds(sid * DZT, DZT)],
                        deg_hbm.at[pl.ds(cid * DPAD + sid * DZT, DZT)])


def _make_seg_sum(with_deg):
    out_type = [jax.ShapeDtypeStruct((NC * NPAD, H), jnp.float32)]
    if with_deg:
        out_type.append(jax.ShapeDtypeStruct((NC * DPAD,), jnp.float32))
    else:
        out_type = out_type[0]
    return pl.kernel(
        functools.partial(_seg_sum_body, with_deg),
        out_type=out_type,
        mesh=plsc.VectorSubcoreMesh(core_axis_name="c", subcore_axis_name="s"),
        compiler_params=pltpu.CompilerParams(use_tc_tiling_on_sc=False),
        scratch_types=[
            pltpu.VMEM((NCH, K), jnp.int32),
            pltpu.VMEM((NCH, K), jnp.int32),
            pltpu.VMEM((NBUF, K, H), jnp.float32),
            pltpu.VMEM((RZ, H), jnp.float32),
            pltpu.VMEM((DZT,), jnp.float32),
            pltpu.VMEM_SHARED((NPAD, H), jnp.float32),
            pltpu.VMEM_SHARED((DPAD,), jnp.float32),
        ] + [pltpu.SemaphoreType.DMA] * (2 * NBUF + 1),
    )


_seg_sum_deg = _make_seg_sum(True)
_seg_sum = _make_seg_sum(False)


def kernel(feats, edge_index, W_self1, W_neigh1, b1, W_self2, W_neigh2, b2):
    src = edge_index[0].reshape(NW, NCH, K)
    dst = edge_index[1].reshape(NW, NCH, K)
    b1r = b1.reshape(1, H)
    b2r = b2.reshape(1, H)

    p1, self1 = _proj2(feats, W_neigh1, W_self1)
    s1_flat, deg_flat = _seg_sum_deg(p1, src, dst)
    s1 = s1_flat.reshape(NC, NPAD, H)
    degp = deg_flat.reshape(NC, DPAD)

    p2, self2 = _combine2(self1, s1, degp, b1r, W_neigh2, W_self2)
    s2_flat = _seg_sum(p2, src, dst)
    s2 = s2_flat.reshape(NC, NPAD, H)

    return _final(self2, s2, degp, b2r)
